# baseline (speedup 1.0000x reference)
import jax
import jax.numpy as jnp
from jax.experimental import pallas as pl
from jax.experimental.pallas import tpu as pltpu

BATCH = 16384
HIDDEN = 256
BLOCK_ROWS = 2048


def _body(bias_ref, a_ref, w_ref, o_ref):
    h = jnp.clip(a_ref[...], 0.0, 1.0)
    o_ref[...] = jnp.dot(h, w_ref[...], preferred_element_type=jnp.float32) + bias_ref[0]


def kernel(accumulator, output_weights, output_bias):
    bias = jnp.reshape(output_bias, (1,)).astype(jnp.float32)
    w2d = jnp.reshape(output_weights, (HIDDEN, 1))
    grid = (BATCH // BLOCK_ROWS,)
    out = pl.pallas_call(
        _body,
        grid=grid,
        in_specs=[
            pl.BlockSpec(memory_space=pltpu.MemorySpace.SMEM),
            pl.BlockSpec((BLOCK_ROWS, HIDDEN), lambda i: (i, 0)),
            pl.BlockSpec((HIDDEN, 1), lambda i: (0, 0)),
        ],
        out_specs=pl.BlockSpec((BLOCK_ROWS, 1), lambda i: (i, 0)),
        out_shape=jax.ShapeDtypeStruct((BATCH, 1), jnp.float32),
    )(bias, accumulator, w2d)
    return jnp.reshape(out, (BATCH,))
